# Initial kernel scaffold; baseline (speedup 1.0000x reference)
import jax
import jax.numpy as jnp
from jax.experimental import pallas as pl


def kernel(rel, arg1, arg2, fact_rel, fact_arg1, fact_arg2, entity_embeddings, W1, W2):
    raise NotImplementedError("write your pallas kernel here")


# speedup vs baseline: 8.0138x; 8.0138x over previous
import functools

import jax
import jax.numpy as jnp
from jax.experimental import pallas as pl

K_TOP = 10


def _pair_kernel(x, xn, y, yn, dot):
    sq = xn[:, None] + yn[None, :] - 2.0 * dot
    return jnp.exp(-jnp.maximum(sq, 0.0) / (2.0 * 256.0))


def _maxmin(body, ke):
    rows = []
    for b in range(body.shape[0]):
        rows.append(jnp.max(jnp.minimum(ke, body[b][None, :]), axis=1))
    return jnp.stack(rows, axis=0)


def _hoppy_body(rel_ref, arg1_ref, arg2_ref, fr_ref, f1_ref, f2_ref,
                ent_ref, w1_ref, w2_ref, sp_ref, po_ref):
    rel = rel_ref[...]
    arg1 = arg1_ref[...]
    arg2 = arg2_ref[...]
    fr = fr_ref[...]
    f1 = f1_ref[...]
    f2 = f2_ref[...]
    ent = ent_ref[...]
    w1 = w1_ref[...]
    w2 = w2_ref[...]

    B = rel.shape[0]
    N = ent.shape[0]

    dot = functools.partial(jnp.dot, preferred_element_type=jnp.float32)

    ent_n = jnp.sum(ent * ent, axis=1)
    fr_n = jnp.sum(fr * fr, axis=1)
    f1_n = jnp.sum(f1 * f1, axis=1)
    f2_n = jnp.sum(f2 * f2, axis=1)
    rel_n = jnp.sum(rel * rel, axis=1)
    arg1_n = jnp.sum(arg1 * arg1, axis=1)
    arg2_n = jnp.sum(arg2 * arg2, axis=1)

    h1 = dot(rel, w1)
    h2 = dot(rel, w2)
    h1_n = jnp.sum(h1 * h1, axis=1)
    h2_n = jnp.sum(h2 * h2, axis=1)

    ke_sp = _pair_kernel(ent, ent_n, f2, f2_n, dot(ent, f2.T))
    ke_po = _pair_kernel(ent, ent_n, f1, f1_n, dot(ent, f1.T))

    k_rel_fr = _pair_kernel(rel, rel_n, fr, fr_n, dot(rel, fr.T))
    k_a1_f1 = _pair_kernel(arg1, arg1_n, f1, f1_n, dot(arg1, f1.T))
    k_a2_f2 = _pair_kernel(arg2, arg2_n, f2, f2_n, dot(arg2, f2.T))
    bh1 = _pair_kernel(h1, h1_n, fr, fr_n, dot(h1, fr.T))
    bh2 = _pair_kernel(h2, h2_n, fr, fr_n, dot(h2, fr.T))

    body_sp0 = jnp.minimum(k_rel_fr, k_a1_f1)
    body_po0 = jnp.minimum(k_rel_fr, k_a2_f2)
    body_s1 = jnp.minimum(bh1, k_a1_f1)
    body_s2 = jnp.minimum(bh2, k_a2_f2)

    s1 = _maxmin(body_s1, ke_sp)
    s2 = _maxmin(body_s2, ke_po)

    col = jax.lax.broadcasted_iota(jnp.int32, (B, N), 1)

    def beam_aggregate(scores, ke_other):
        cur = scores
        g = None
        for _ in range(K_TOP):
            m = jnp.max(cur, axis=1, keepdims=True)
            sel_idx = jnp.min(jnp.where(cur == m, col, N), axis=1,
                              keepdims=True)
            sel = (col == sel_idx)
            kz = dot(sel.astype(jnp.float32), ke_other)
            contrib = jnp.minimum(kz, m)
            g = contrib if g is None else jnp.maximum(g, contrib)
            cur = jnp.where(sel, -jnp.inf, cur)
        return g

    g_sp = beam_aggregate(s1, ke_po)
    g_po = beam_aggregate(s2, ke_sp)

    cbody_sp = jnp.maximum(body_sp0, jnp.minimum(bh2, g_sp))
    cbody_po = jnp.maximum(body_po0, jnp.minimum(bh1, g_po))

    sp_ref[...] = _maxmin(cbody_sp, ke_sp)
    po_ref[...] = _maxmin(cbody_po, ke_po)


def kernel(rel, arg1, arg2, fact_rel, fact_arg1, fact_arg2,
           entity_embeddings, W1, W2):
    B = rel.shape[0]
    N = entity_embeddings.shape[0]
    out = pl.pallas_call(
        _hoppy_body,
        out_shape=(
            jax.ShapeDtypeStruct((B, N), jnp.float32),
            jax.ShapeDtypeStruct((B, N), jnp.float32),
        ),
    )(rel, arg1, arg2, fact_rel, fact_arg1, fact_arg2,
      entity_embeddings, W1, W2)
    return out


# speedup vs baseline: 8.8919x; 1.1096x over previous
import jax
import jax.numpy as jnp
from jax.experimental import pallas as pl

K_TOP = 10


def _dotT(x, y):
    return jax.lax.dot_general(x, y, (((1,), (1,)), ((), ())),
                               preferred_element_type=jnp.float32)


def _dot(x, y):
    return jax.lax.dot_general(x, y, (((1,), (0,)), ((), ())),
                               preferred_element_type=jnp.float32)


def _pair_kernel(x, xn, y, yn, dot_xy):
    sq = xn[:, None] + yn[None, :] - 2.0 * dot_xy
    return jnp.exp(-jnp.maximum(sq, 0.0) / (2.0 * 256.0))


def _maxminT(bodyT, keT):
    rows = []
    for b in range(bodyT.shape[1]):
        rows.append(jnp.max(jnp.minimum(keT, bodyT[:, b:b + 1]), axis=0,
                            keepdims=True))
    return jnp.concatenate(rows, axis=0)


def _hoppy_body(rel_ref, arg1_ref, arg2_ref, fr_ref, f1_ref, f2_ref,
                ent_ref, w1_ref, w2_ref, sp_ref, po_ref):
    rel = rel_ref[...]
    arg1 = arg1_ref[...]
    arg2 = arg2_ref[...]
    fr = fr_ref[...]
    f1 = f1_ref[...]
    f2 = f2_ref[...]
    ent = ent_ref[...]
    w1 = w1_ref[...]
    w2 = w2_ref[...]

    B = rel.shape[0]
    N = ent.shape[0]

    ent_n = jnp.sum(ent * ent, axis=1)
    fr_n = jnp.sum(fr * fr, axis=1)
    f1_n = jnp.sum(f1 * f1, axis=1)
    f2_n = jnp.sum(f2 * f2, axis=1)
    rel_n = jnp.sum(rel * rel, axis=1)
    arg1_n = jnp.sum(arg1 * arg1, axis=1)
    arg2_n = jnp.sum(arg2 * arg2, axis=1)

    h1 = _dot(rel, w1)
    h2 = _dot(rel, w2)
    h1_n = jnp.sum(h1 * h1, axis=1)
    h2_n = jnp.sum(h2 * h2, axis=1)

    keT_sp = _pair_kernel(f2, f2_n, ent, ent_n, _dotT(f2, ent))
    keT_po = _pair_kernel(f1, f1_n, ent, ent_n, _dotT(f1, ent))

    kT_rel_fr = _pair_kernel(fr, fr_n, rel, rel_n, _dotT(fr, rel))
    kT_a1_f1 = _pair_kernel(f1, f1_n, arg1, arg1_n, _dotT(f1, arg1))
    kT_a2_f2 = _pair_kernel(f2, f2_n, arg2, arg2_n, _dotT(f2, arg2))
    bhT1 = _pair_kernel(fr, fr_n, h1, h1_n, _dotT(fr, h1))
    bhT2 = _pair_kernel(fr, fr_n, h2, h2_n, _dotT(fr, h2))

    bodyT_sp0 = jnp.minimum(kT_rel_fr, kT_a1_f1)
    bodyT_po0 = jnp.minimum(kT_rel_fr, kT_a2_f2)
    bodyT_s1 = jnp.minimum(bhT1, kT_a1_f1)
    bodyT_s2 = jnp.minimum(bhT2, kT_a2_f2)

    s1 = _maxminT(bodyT_s1, keT_sp)
    s2 = _maxminT(bodyT_s2, keT_po)

    col = jax.lax.broadcasted_iota(jnp.int32, (B, N), 1)
    rowT = jax.lax.broadcasted_iota(jnp.int32, (N, B), 0)

    def beam_aggregate(scores, keT_other):
        cur = scores
        g = None
        for _ in range(K_TOP):
            m = jnp.max(cur, axis=1, keepdims=True)
            sel_idx = jnp.min(jnp.where(cur == m, col, N), axis=1,
                              keepdims=True)
            selT = (rowT == sel_idx.reshape(1, B))
            kzT = _dot(keT_other, selT.astype(jnp.float32))
            contrib = jnp.minimum(kzT, m.reshape(1, B))
            g = contrib if g is None else jnp.maximum(g, contrib)
            cur = jnp.where(col == sel_idx, -jnp.inf, cur)
        return g

    gT_sp = beam_aggregate(s1, keT_po)
    gT_po = beam_aggregate(s2, keT_sp)

    cbodyT_sp = jnp.maximum(bodyT_sp0, jnp.minimum(bhT2, gT_sp))
    cbodyT_po = jnp.maximum(bodyT_po0, jnp.minimum(bhT1, gT_po))

    sp_ref[...] = _maxminT(cbodyT_sp, keT_sp)
    po_ref[...] = _maxminT(cbodyT_po, keT_po)


def kernel(rel, arg1, arg2, fact_rel, fact_arg1, fact_arg2,
           entity_embeddings, W1, W2):
    B = rel.shape[0]
    N = entity_embeddings.shape[0]
    out = pl.pallas_call(
        _hoppy_body,
        out_shape=(
            jax.ShapeDtypeStruct((B, N), jnp.float32),
            jax.ShapeDtypeStruct((B, N), jnp.float32),
        ),
    )(rel, arg1, arg2, fact_rel, fact_arg1, fact_arg2,
      entity_embeddings, W1, W2)
    return out
